# baseline (speedup 1.0000x reference)
import functools

import jax
import jax.numpy as jnp
from jax import lax
from jax.experimental import pallas as pl
from jax.experimental.pallas import tpu as pltpu

N = 20000
K = 4000
NPAD = 4096
B = 512
NB = NPAD // B
Q = 1000
QPAD = 1024
IOU_T = 0.3
NEG = -1e9
NGROUP = 4

L = 128
N_ROWS = 160


def _canon_rows(raw):
    cx = raw[0:1, :] * 1024.0
    cy = raw[1:2, :] * 1024.0
    w = raw[2:3, :] * 64.0 + 1.0
    h = raw[3:4, :] * 64.0 + 1.0
    x1 = cx - w / 2
    y1 = cy - h / 2
    x2 = cx + w / 2
    y2 = cy + h / 2
    return x1, y1, x2, y2, (x2 - x1) * (y2 - y1)


def _iou_gt(cols, rows):
    bx1, by1, bx2, by2, ba = cols
    x1r, y1r, x2r, y2r, ar = rows
    ix1 = jnp.maximum(bx1, x1r)
    iy1 = jnp.maximum(by1, y1r)
    ix2 = jnp.minimum(bx2, x2r)
    iy2 = jnp.minimum(by2, y2r)
    iw = jnp.maximum(ix2 - ix1, 0.0)
    ih = jnp.maximum(iy2 - iy1, 0.0)
    inter = iw * ih
    union = ba + ar - inter
    return (inter > IOU_T * union).astype(jnp.float32)


def _canon_cols(raw):
    cx = raw[:, 0:1] * 1024.0
    cy = raw[:, 1:2] * 1024.0
    w = raw[:, 2:3] * 64.0 + 1.0
    h = raw[:, 3:4] * 64.0 + 1.0
    x1 = cx - w / 2
    y1 = cy - h / 2
    x2 = cx + w / 2
    y2 = cy + h / 2
    return x1, y1, x2, y2, (x2 - x1) * (y2 - y1)


def _nms_body(tbl_ref, idx_ref, sc_ref, out_ref, rawc_ref, sup_ref, keep_ref, acc_ref):
    f32 = jnp.float32

    sup_ref[...] = jnp.zeros((NB, B), f32)
    keep_ref[...] = jnp.zeros((NB, B), f32)
    iota_l = lax.broadcasted_iota(jnp.int32, (1, B), 1)
    il = lax.broadcasted_iota(jnp.int32, (B, B), 0)
    jl = lax.broadcasted_iota(jnp.int32, (B, B), 1)
    tri = (il < jl).astype(f32)
    eye = (il == jl).astype(f32)

    iota_w = lax.broadcasted_iota(jnp.int32, (1, N_ROWS), 1)
    iota_L = lax.broadcasted_iota(jnp.int32, (1, L), 1)
    for k in range(NPAD // L):
        idxb = idx_ref[k * L:(k + 1) * L, :]
        hi = idxb // L
        lo = idxb - hi * L
        eq1 = (hi == iota_w).astype(f32)
        rowv = lax.dot_general(eq1, tbl_ref[...], (((1,), (0,)), ((), ())),
                               precision=lax.Precision.HIGHEST,
                               preferred_element_type=f32)
        eq2 = (lo == iota_L).astype(f32)
        rawc_ref[k * L:(k + 1) * L, :] = jnp.concatenate([
            jnp.sum(rowv[:, c * L:(c + 1) * L] * eq2, axis=1, keepdims=True)
            for c in range(4)], axis=1)

    rawr = jnp.concatenate([
        lax.dot_general(rawc_ref[k * B:(k + 1) * B, :], eye,
                        (((0,), (0,)), ((), ())),
                        precision=lax.Precision.HIGHEST,
                        preferred_element_type=f32)
        for k in range(NB)], axis=1)
    rows_all = _canon_rows(rawr)
    x1r, y1r, x2r, y2r, _ = rows_all

    def block_step(k, nk):
        c0 = k * B

        @pl.when(nk < float(Q))
        def _process():
            braw = rawc_ref[pl.ds(c0, B), :]
            cols = _canon_cols(braw)
            brows = tuple(
                lax.dot_general(v, eye, (((0,), (0,)), ((), ())),
                                precision=lax.Precision.HIGHEST,
                                preferred_element_type=f32)
                for v in cols)
            S_tri = _iou_gt(cols, brows) * tri

            a0 = 1.0 - sup_ref[pl.ds(k, 1), :]

            def fstep(kk):
                cnt = jnp.dot(kk, S_tri, preferred_element_type=f32)
                return a0 * (cnt < 0.5).astype(f32)

            def w_cond(c):
                kprev, kk = c
                return jnp.any(kprev != kk)

            def w_body(c):
                _, kk = c
                return kk, fstep(kk)

            k1 = fstep(a0)
            k2 = fstep(k1)
            _, a = lax.while_loop(w_cond, w_body, (k1, k2))

            keep_ref[pl.ds(k, 1), :] = a
            GW = NPAD // NGROUP
            GB = GW // B
            for g in range(NGROUP):

                @pl.when(g >= k // GB)
                def _prop(g=g):
                    rows_g = tuple(v[0:1, g * GW:(g + 1) * GW]
                                   for v in rows_all)
                    S_g = _iou_gt(cols, rows_g)
                    cnt = jnp.dot(a, S_g, preferred_element_type=f32)
                    hit = (cnt > 0.0).astype(f32)
                    for m in range(GB):
                        row = g * GB + m
                        sup_ref[row:row + 1, :] = jnp.maximum(
                            sup_ref[row:row + 1, :],
                            hit[0:1, m * B:(m + 1) * B])

        real_row = ((iota_l + c0) < K).astype(f32)
        return nk + jnp.sum(keep_ref[pl.ds(k, 1), :] * real_row)

    lax.fori_loop(0, NB, block_step, jnp.float32(0.0))

    keep_rows = keep_ref[...]

    gidx = (lax.broadcasted_iota(jnp.int32, (NB, B), 0) * B
            + lax.broadcasted_iota(jnp.int32, (NB, B), 1))
    real = (gidx < K).astype(f32)
    alive = keep_rows * real
    dead = (1.0 - keep_rows) * real

    Texc = tri
    ir = lax.broadcasted_iota(jnp.int32, (NB, NB), 0)
    jr = lax.broadcasted_iota(jnp.int32, (NB, NB), 1)
    Trow = (jr < ir).astype(f32)
    ones_col = jnp.ones((B, 1), f32)

    def excl_rank(m):
        within = jnp.dot(m, Texc, preferred_element_type=f32)
        rowsum = jnp.dot(m, ones_col, preferred_element_type=f32)
        offs = jnp.dot(Trow, rowsum, preferred_element_type=f32)
        return within + offs, jnp.sum(rowsum)

    rank_keep, n_keep = excl_rank(alive)
    rank_dead, _ = excl_rank(dead)
    r = jnp.where(alive > 0.0, rank_keep,
                  jnp.where(dead > 0.0, n_keep + rank_dead, 2.0 * NPAD))

    iq = lax.broadcasted_iota(jnp.int32, (QPAD, 1), 0).astype(f32)
    acc_ref[...] = jnp.zeros((QPAD, 8), f32)
    for k in range(NB):
        rk = r[k:k + 1, :]

        @pl.when(jnp.min(rk) < float(Q))
        def _select(k=k, rk=rk):
            alv = alive[k:k + 1, :]
            sck = sc_ref[0:1, k * B:(k + 1) * B]
            msk = jnp.where(alv > 0.0, sck, NEG)
            vk = jnp.concatenate([
                msk,
                x1r[0:1, k * B:(k + 1) * B],
                y1r[0:1, k * B:(k + 1) * B],
                x2r[0:1, k * B:(k + 1) * B],
                y2r[0:1, k * B:(k + 1) * B],
                jnp.zeros((3, B), f32),
            ], axis=0)
            eq = (iq == rk).astype(f32)
            acc_ref[...] = acc_ref[...] + lax.dot_general(
                eq, vk, (((1,), (1,)), ((), ())),
                precision=lax.Precision.HIGHEST,
                preferred_element_type=f32)
    out_ref[...] = acc_ref[...]


def _nms_call(tbl, idx, sc, interpret=False):
    return pl.pallas_call(
        _nms_body,
        out_shape=jax.ShapeDtypeStruct((QPAD, 8), jnp.float32),
        scratch_shapes=[
            pltpu.VMEM((NPAD, 4), jnp.float32),
            pltpu.VMEM((NB, B), jnp.float32),
            pltpu.VMEM((NB, B), jnp.float32),
            pltpu.VMEM((QPAD, 8), jnp.float32),
        ],
        interpret=interpret,
    )(tbl, idx, sc)


@functools.partial(jax.jit, static_argnames=("interpret",))
def _run(boxes, scores, interpret=False):
    top_scores, top_idx = lax.top_k(scores, NPAD)
    tbl = jnp.pad(boxes, ((0, N_ROWS * L - N), (0, 0)))
    tbl = tbl.reshape(N_ROWS, L, 4).transpose(0, 2, 1).reshape(N_ROWS, 4 * L)
    idx = top_idx.reshape(NPAD, 1)
    sc = top_scores.reshape(1, NPAD)
    out = _nms_call(tbl, idx, sc, interpret=interpret)
    return out[:Q, :5]


def kernel(boxes, scores):
    return _run(boxes, scores)


# speedup vs baseline: 1.0087x; 1.0087x over previous
import functools

import jax
import jax.numpy as jnp
from jax import lax
from jax.experimental import pallas as pl
from jax.experimental.pallas import tpu as pltpu

N = 20000
K = 4000
NPAD = 4096
B = 256
NB = NPAD // B
Q = 1000
QPAD = 1024
IOU_T = 0.3
NEG = -1e9
NGROUP = 4

L = 128
N_ROWS = 160


def _canon_rows(raw):
    cx = raw[0:1, :] * 1024.0
    cy = raw[1:2, :] * 1024.0
    w = raw[2:3, :] * 64.0 + 1.0
    h = raw[3:4, :] * 64.0 + 1.0
    x1 = cx - w / 2
    y1 = cy - h / 2
    x2 = cx + w / 2
    y2 = cy + h / 2
    return x1, y1, x2, y2, (x2 - x1) * (y2 - y1)


def _iou_gt(cols, rows):
    bx1, by1, bx2, by2, ba = cols
    x1r, y1r, x2r, y2r, ar = rows
    ix1 = jnp.maximum(bx1, x1r)
    iy1 = jnp.maximum(by1, y1r)
    ix2 = jnp.minimum(bx2, x2r)
    iy2 = jnp.minimum(by2, y2r)
    iw = jnp.maximum(ix2 - ix1, 0.0)
    ih = jnp.maximum(iy2 - iy1, 0.0)
    inter = iw * ih
    union = ba + ar - inter
    return (inter > IOU_T * union).astype(jnp.float32)


def _canon_cols(raw):
    cx = raw[:, 0:1] * 1024.0
    cy = raw[:, 1:2] * 1024.0
    w = raw[:, 2:3] * 64.0 + 1.0
    h = raw[:, 3:4] * 64.0 + 1.0
    x1 = cx - w / 2
    y1 = cy - h / 2
    x2 = cx + w / 2
    y2 = cy + h / 2
    return x1, y1, x2, y2, (x2 - x1) * (y2 - y1)


def _nms_body(tbl_ref, idx_ref, sc_ref, out_ref, rawc_ref, sup_ref, keep_ref, acc_ref):
    f32 = jnp.float32

    sup_ref[...] = jnp.zeros((NB, B), f32)
    keep_ref[...] = jnp.zeros((NB, B), f32)
    iota_l = lax.broadcasted_iota(jnp.int32, (1, B), 1)
    il = lax.broadcasted_iota(jnp.int32, (B, B), 0)
    jl = lax.broadcasted_iota(jnp.int32, (B, B), 1)
    tri = (il < jl).astype(f32)
    eye = (il == jl).astype(f32)

    iota_w = lax.broadcasted_iota(jnp.int32, (1, N_ROWS), 1)
    iota_L = lax.broadcasted_iota(jnp.int32, (1, L), 1)
    for k in range(NPAD // L):
        idxb = idx_ref[k * L:(k + 1) * L, :]
        hi = idxb // L
        lo = idxb - hi * L
        eq1 = (hi == iota_w).astype(f32)
        rowv = lax.dot_general(eq1, tbl_ref[...], (((1,), (0,)), ((), ())),
                               precision=lax.Precision.HIGHEST,
                               preferred_element_type=f32)
        eq2 = (lo == iota_L).astype(f32)
        rawc_ref[k * L:(k + 1) * L, :] = jnp.concatenate([
            jnp.sum(rowv[:, c * L:(c + 1) * L] * eq2, axis=1, keepdims=True)
            for c in range(4)], axis=1)

    rawr = jnp.concatenate([
        lax.dot_general(rawc_ref[k * B:(k + 1) * B, :], eye,
                        (((0,), (0,)), ((), ())),
                        precision=lax.Precision.HIGHEST,
                        preferred_element_type=f32)
        for k in range(NB)], axis=1)
    rows_all = _canon_rows(rawr)
    x1r, y1r, x2r, y2r, _ = rows_all

    def block_step(k, nk):
        c0 = k * B

        @pl.when(nk < float(Q))
        def _process():
            braw = rawc_ref[pl.ds(c0, B), :]
            cols = _canon_cols(braw)
            brows = tuple(
                lax.dot_general(v, eye, (((0,), (0,)), ((), ())),
                                precision=lax.Precision.HIGHEST,
                                preferred_element_type=f32)
                for v in cols)
            S_tri = _iou_gt(cols, brows) * tri

            a0 = 1.0 - sup_ref[pl.ds(k, 1), :]

            def fstep(kk):
                cnt = jnp.dot(kk, S_tri, preferred_element_type=f32)
                return a0 * (cnt < 0.5).astype(f32)

            def w_cond(c):
                kprev, kk = c
                return jnp.any(kprev != kk)

            def w_body(c):
                _, kk = c
                return kk, fstep(kk)

            k1 = fstep(a0)
            k2 = fstep(k1)
            _, a = lax.while_loop(w_cond, w_body, (k1, k2))

            keep_ref[pl.ds(k, 1), :] = a
            GW = NPAD // NGROUP
            GB = GW // B
            for g in range(NGROUP):

                @pl.when(g >= k // GB)
                def _prop(g=g):
                    rows_g = tuple(v[0:1, g * GW:(g + 1) * GW]
                                   for v in rows_all)
                    S_g = _iou_gt(cols, rows_g)
                    cnt = jnp.dot(a, S_g, preferred_element_type=f32)
                    hit = (cnt > 0.0).astype(f32)
                    for m in range(GB):
                        row = g * GB + m
                        sup_ref[row:row + 1, :] = jnp.maximum(
                            sup_ref[row:row + 1, :],
                            hit[0:1, m * B:(m + 1) * B])

        real_row = ((iota_l + c0) < K).astype(f32)
        return nk + jnp.sum(keep_ref[pl.ds(k, 1), :] * real_row)

    lax.fori_loop(0, NB, block_step, jnp.float32(0.0))

    keep_rows = keep_ref[...]

    gidx = (lax.broadcasted_iota(jnp.int32, (NB, B), 0) * B
            + lax.broadcasted_iota(jnp.int32, (NB, B), 1))
    real = (gidx < K).astype(f32)
    alive = keep_rows * real
    dead = (1.0 - keep_rows) * real

    Texc = tri
    ir = lax.broadcasted_iota(jnp.int32, (NB, NB), 0)
    jr = lax.broadcasted_iota(jnp.int32, (NB, NB), 1)
    Trow = (jr < ir).astype(f32)
    ones_col = jnp.ones((B, 1), f32)

    def excl_rank(m):
        within = jnp.dot(m, Texc, preferred_element_type=f32)
        rowsum = jnp.dot(m, ones_col, preferred_element_type=f32)
        offs = jnp.dot(Trow, rowsum, preferred_element_type=f32)
        return within + offs, jnp.sum(rowsum)

    rank_keep, n_keep = excl_rank(alive)
    rank_dead, _ = excl_rank(dead)
    r = jnp.where(alive > 0.0, rank_keep,
                  jnp.where(dead > 0.0, n_keep + rank_dead, 2.0 * NPAD))

    iq = lax.broadcasted_iota(jnp.int32, (QPAD, 1), 0).astype(f32)
    acc_ref[...] = jnp.zeros((QPAD, 8), f32)
    for k in range(NB):
        rk = r[k:k + 1, :]

        @pl.when(jnp.min(rk) < float(Q))
        def _select(k=k, rk=rk):
            alv = alive[k:k + 1, :]
            sck = sc_ref[0:1, k * B:(k + 1) * B]
            msk = jnp.where(alv > 0.0, sck, NEG)
            vk = jnp.concatenate([
                msk,
                x1r[0:1, k * B:(k + 1) * B],
                y1r[0:1, k * B:(k + 1) * B],
                x2r[0:1, k * B:(k + 1) * B],
                y2r[0:1, k * B:(k + 1) * B],
                jnp.zeros((3, B), f32),
            ], axis=0)
            eq = (iq == rk).astype(f32)
            acc_ref[...] = acc_ref[...] + lax.dot_general(
                eq, vk, (((1,), (1,)), ((), ())),
                precision=lax.Precision.HIGHEST,
                preferred_element_type=f32)
    out_ref[...] = acc_ref[...]


def _nms_call(tbl, idx, sc, interpret=False):
    return pl.pallas_call(
        _nms_body,
        out_shape=jax.ShapeDtypeStruct((QPAD, 8), jnp.float32),
        scratch_shapes=[
            pltpu.VMEM((NPAD, 4), jnp.float32),
            pltpu.VMEM((NB, B), jnp.float32),
            pltpu.VMEM((NB, B), jnp.float32),
            pltpu.VMEM((QPAD, 8), jnp.float32),
        ],
        interpret=interpret,
    )(tbl, idx, sc)


@functools.partial(jax.jit, static_argnames=("interpret",))
def _run(boxes, scores, interpret=False):
    top_scores, top_idx = lax.top_k(scores, NPAD)
    tbl = jnp.pad(boxes, ((0, N_ROWS * L - N), (0, 0)))
    tbl = tbl.reshape(N_ROWS, L, 4).transpose(0, 2, 1).reshape(N_ROWS, 4 * L)
    idx = top_idx.reshape(NPAD, 1)
    sc = top_scores.reshape(1, NPAD)
    out = _nms_call(tbl, idx, sc, interpret=interpret)
    return out[:Q, :5]


def kernel(boxes, scores):
    return _run(boxes, scores)
